# baseline (speedup 1.0000x reference)
import functools

import jax
import jax.numpy as jnp
from jax import lax
from jax.experimental import pallas as pl
from jax.experimental.pallas import tpu as pltpu
from jax.experimental.pallas import tpu_sc as plsc

B = 819200
D = 64
NC = 2
NS = 16
NW = NC * NS
CHUNK = 128
NCH = B // (NW * CHUNK)
NTC = B // CHUNK
NBUF = 2


def _make_kernel():
    mesh = plsc.VectorSubcoreMesh(core_axis_name="c", subcore_axis_name="s")

    @functools.partial(
        pl.kernel,
        mesh=mesh,
        out_type=jax.ShapeDtypeStruct((8, NTC, 8, CHUNK), jnp.float32),
        scratch_types=[
            pltpu.VMEM((NCH, CHUNK), jnp.int32),
            pltpu.VMEM((NBUF, CHUNK, D), jnp.float32),
            pltpu.VMEM((NBUF, 8, 1, 8, CHUNK), jnp.float32),
        ] + [pltpu.SemaphoreType.DMA] * (2 * NBUF),
        compiler_params=pltpu.CompilerParams(
            use_tc_tiling_on_sc=False, needs_layout_passes=False
        ),
    )
    def emb(x_hbm, tab_hbm, out_hbm, idx_v, g_v, t_v, *sems):
        gsem = sems[:NBUF]
        osem = sems[NBUF:]
        wid = lax.axis_index("s") * NC + lax.axis_index("c")
        tc0 = wid * NCH
        pltpu.sync_copy(x_hbm.at[pl.ds(tc0, NCH)], idx_v)

        iota = lax.iota(jnp.int32, 16)
        jvecs = [g8 * 16 + iota for g8 in range(8)]

        for b in range(NBUF):
            pltpu.async_copy(tab_hbm.at[idx_v.at[b]], g_v.at[b], gsem[b])

        def transpose_chunk(b):
            g2 = g_v.at[b]

            def tr_body(tr, carry):
                for s in range(8):
                    d = tr * 8 + s
                    dvec = jnp.zeros((16,), jnp.int32) + d
                    for g8 in range(8):
                        v = plsc.load_gather(g2, [jvecs[g8], dvec])
                        t_v[b, tr, 0, s, pl.ds(g8 * 16, 16)] = v
                return carry

            lax.fori_loop(0, 8, tr_body, 0)

        def outer(g, carry):
            for b in range(NBUF):
                j = g * NBUF + b
                pltpu.make_async_copy(
                    tab_hbm.at[pl.ds(0, CHUNK)], g_v.at[b], gsem[b]
                ).wait()
                @pl.when(j >= NBUF)
                def _():
                    pltpu.make_async_copy(
                        t_v.at[b], out_hbm.at[pl.ds(0, 8), pl.ds(0, 1)], osem[b]
                    ).wait()

                transpose_chunk(b)

                pltpu.async_copy(
                    t_v.at[b],
                    out_hbm.at[pl.ds(0, 8), pl.ds(tc0 + j, 1)],
                    osem[b],
                )

                @pl.when(j + NBUF < NCH)
                def _():
                    pltpu.async_copy(
                        tab_hbm.at[idx_v.at[j + NBUF]], g_v.at[b], gsem[b]
                    )
            return carry

        lax.fori_loop(0, NCH // NBUF, outer, 0)

        for b in range(NBUF):
            pltpu.make_async_copy(
                t_v.at[b], out_hbm.at[pl.ds(0, 8), pl.ds(0, 1)], osem[b]
            ).wait()

    return emb


_emb = _make_kernel()


def kernel(x, table):
    x2 = x.reshape(NTC, CHUNK).astype(jnp.int32)
    out4d = _emb(x2, table)
    return out4d.transpose(1, 3, 0, 2).reshape(B, D)


# speedup vs baseline: 1.2897x; 1.2897x over previous
import functools

import jax
import jax.numpy as jnp
from jax import lax
from jax.experimental import pallas as pl
from jax.experimental.pallas import tpu as pltpu
from jax.experimental.pallas import tpu_sc as plsc

B = 819200
D = 64
NC = 2
NS = 16
NW = NC * NS
CHUNK = 128
NCH = B // (NW * CHUNK)
NTC = B // CHUNK
NBUF = 4


def _make_kernel():
    mesh = plsc.VectorSubcoreMesh(core_axis_name="c", subcore_axis_name="s")

    @functools.partial(
        pl.kernel,
        mesh=mesh,
        out_type=jax.ShapeDtypeStruct((8, NTC, 8 * CHUNK), jnp.float32),
        scratch_types=[
            pltpu.VMEM((NCH, CHUNK), jnp.int32),
            pltpu.VMEM((NBUF, CHUNK, D), jnp.float32),
            pltpu.VMEM((NBUF, 8, 1, 8 * CHUNK), jnp.float32),
        ] + [pltpu.SemaphoreType.DMA] * (2 * NBUF),
        compiler_params=pltpu.CompilerParams(
            use_tc_tiling_on_sc=False, needs_layout_passes=False
        ),
    )
    def emb(x_hbm, tab_hbm, out_hbm, idx_v, g_v, t_v, *sems):
        gsem = sems[:NBUF]
        osem = sems[NBUF:]
        wid = lax.axis_index("s") * NC + lax.axis_index("c")
        tc0 = wid * NCH
        pltpu.sync_copy(x_hbm.at[pl.ds(tc0, NCH)], idx_v)

        iota = lax.iota(jnp.int32, 16)
        jvecs = [g8 * 16 + iota for g8 in range(8)]
        zero16 = iota * 0

        for b in range(NBUF):
            pltpu.async_copy(tab_hbm.at[idx_v.at[b]], g_v.at[b], gsem[b])

        def transpose_chunk(b):
            g2 = g_v.at[b]
            t3 = t_v.at[b]

            @plsc.parallel_loop(0, 8, 1, unroll=2)
            def tr_body(tr):
                for s in range(8):
                    dvec = zero16 + (tr * 8 + s)
                    for g8 in range(8):
                        v = plsc.load_gather(g2, [jvecs[g8], dvec])
                        t3[tr, 0, pl.ds(s * CHUNK + g8 * 16, 16)] = v

        def outer(g, carry):
            for b in range(NBUF):
                j = g * NBUF + b
                pltpu.make_async_copy(
                    tab_hbm.at[pl.ds(0, CHUNK)], g_v.at[b], gsem[b]
                ).wait()
                @pl.when(j >= NBUF)
                def _():
                    pltpu.make_async_copy(
                        t_v.at[b], out_hbm.at[pl.ds(0, 8), pl.ds(0, 1)], osem[b]
                    ).wait()

                transpose_chunk(b)

                pltpu.async_copy(
                    t_v.at[b],
                    out_hbm.at[pl.ds(0, 8), pl.ds(tc0 + j, 1)],
                    osem[b],
                )

                @pl.when(j + NBUF < NCH)
                def _():
                    pltpu.async_copy(
                        tab_hbm.at[idx_v.at[j + NBUF]], g_v.at[b], gsem[b]
                    )
            return carry

        lax.fori_loop(0, NCH // NBUF, outer, 0)

        for b in range(NBUF):
            pltpu.make_async_copy(
                t_v.at[b], out_hbm.at[pl.ds(0, 8), pl.ds(0, 1)], osem[b]
            ).wait()

    return emb


_emb = _make_kernel()


def kernel(x, table):
    x2 = x.reshape(NTC, CHUNK).astype(jnp.int32)
    out3d = _emb(x2, table)
    return (
        out3d.reshape(8, NTC, 8, CHUNK)
        .transpose(1, 3, 0, 2)
        .reshape(B, D)
    )
